# baseline (speedup 1.0000x reference)
import functools

import jax
import jax.numpy as jnp
from jax import lax
from jax.experimental import pallas as pl
from jax.experimental.pallas import tpu as pltpu
from jax.experimental.pallas import tpu_sc as plsc

_NUM_CORES = 2
_NUM_SUBCORES = 16
_NUM_WORKERS = _NUM_CORES * _NUM_SUBCORES


@functools.lru_cache(maxsize=None)
def _make_gather(T, Bm, D):
    bw = Bm // _NUM_WORKERS
    mesh = plsc.VectorSubcoreMesh(core_axis_name="c", subcore_axis_name="s")

    @functools.partial(
        pl.kernel,
        mesh=mesh,
        out_type=jax.ShapeDtypeStruct((T, Bm, D), jnp.float32),
        compiler_params=pltpu.CompilerParams(use_tc_tiling_on_sc=False),
        scratch_types=[
            pltpu.VMEM((T * bw,), jnp.int32),
            pltpu.VMEM((2, bw, D), jnp.float32),
            pltpu.SemaphoreType.DMA,
            pltpu.SemaphoreType.DMA,
            pltpu.SemaphoreType.DMA,
            pltpu.SemaphoreType.DMA,
        ],
    )
    def gather_kernel(idx_hbm, table_hbm, out_hbm, idx_v, rows_v,
                      gsem0, gsem1, ssem0, ssem1):
        wid = lax.axis_index("s") * _NUM_CORES + lax.axis_index("c")
        b0 = wid * bw
        gsem = (gsem0, gsem1)
        ssem = (ssem0, ssem1)

        for t in range(T):
            pltpu.async_copy(
                idx_hbm.at[t, pl.ds(b0, bw)],
                idx_v.at[pl.ds(t * bw, bw)], gsem0)
        for t in range(T):
            pltpu.make_async_copy(
                idx_hbm.at[t, pl.ds(b0, bw)],
                idx_v.at[pl.ds(t * bw, bw)], gsem0).wait()

        gathers = [None] * T
        stores = [None] * T
        for t in range(T):
            b = t % 2
            if t >= 2:
                stores[t - 2].wait()
            gathers[t] = pltpu.async_copy(
                table_hbm.at[idx_v.at[pl.ds(t * bw, bw)]],
                rows_v.at[b], gsem[b])
            if t >= 1:
                gathers[t - 1].wait()
                stores[t - 1] = pltpu.async_copy(
                    rows_v.at[(t - 1) % 2],
                    out_hbm.at[t - 1, pl.ds(b0, bw)],
                    ssem[(t - 1) % 2])
        t = T - 1
        gathers[t].wait()
        stores[t] = pltpu.async_copy(
            rows_v.at[t % 2],
            out_hbm.at[t, pl.ds(b0, bw)], ssem[t % 2])
        stores[t - 1].wait()
        stores[t].wait()

    return gather_kernel


def kernel(input, weight):
    Bm, T = input.shape
    D = weight.shape[1]
    idx_t = jnp.transpose(input).astype(jnp.int32)
    out = _make_gather(T, Bm, D)(idx_t, weight)
    return jnp.transpose(out, (1, 0, 2))


# speedup vs baseline: 1.0035x; 1.0035x over previous
import functools

import jax
import jax.numpy as jnp
from jax import lax
from jax.experimental import pallas as pl
from jax.experimental.pallas import tpu as pltpu
from jax.experimental.pallas import tpu_sc as plsc

_NUM_CORES = 2
_NUM_SUBCORES = 16
_NUM_WORKERS = _NUM_CORES * _NUM_SUBCORES
_CHUNK = 1280


@functools.lru_cache(maxsize=None)
def _make_gather(B, D):
    b_per_w = B // _NUM_WORKERS
    n_chunks = b_per_w // _CHUNK
    mesh = plsc.VectorSubcoreMesh(core_axis_name="c", subcore_axis_name="s")

    @functools.partial(
        pl.kernel,
        mesh=mesh,
        out_type=jax.ShapeDtypeStruct((B, D), jnp.float32),
        compiler_params=pltpu.CompilerParams(use_tc_tiling_on_sc=False),
        scratch_types=[
            pltpu.VMEM((b_per_w,), jnp.int32),
            pltpu.VMEM((2, _CHUNK, D), jnp.float32),
            pltpu.SemaphoreType.DMA,
            pltpu.SemaphoreType.DMA,
            pltpu.SemaphoreType.DMA,
            pltpu.SemaphoreType.DMA,
        ],
    )
    def gather_kernel(idx_hbm, table_hbm, out_hbm, idx_v, rows_v,
                      gsem0, gsem1, ssem0, ssem1):
        wid = lax.axis_index("s") * _NUM_CORES + lax.axis_index("c")
        base = wid * b_per_w
        gsem = (gsem0, gsem1)
        ssem = (ssem0, ssem1)

        pltpu.sync_copy(idx_hbm.at[pl.ds(base, b_per_w)], idx_v)

        gathers = [None] * n_chunks
        stores = [None] * n_chunks
        for g in range(n_chunks):
            b = g % 2
            if g >= 2:
                stores[g - 2].wait()
            gathers[g] = pltpu.async_copy(
                table_hbm.at[idx_v.at[pl.ds(g * _CHUNK, _CHUNK)]],
                rows_v.at[b], gsem[b])
            if g >= 1:
                gathers[g - 1].wait()
                stores[g - 1] = pltpu.async_copy(
                    rows_v.at[(g - 1) % 2],
                    out_hbm.at[pl.ds(base + (g - 1) * _CHUNK, _CHUNK)],
                    ssem[(g - 1) % 2])
        g = n_chunks - 1
        gathers[g].wait()
        stores[g] = pltpu.async_copy(
            rows_v.at[g % 2],
            out_hbm.at[pl.ds(base + g * _CHUNK, _CHUNK)], ssem[g % 2])
        stores[g - 1].wait()
        stores[g].wait()

    return gather_kernel


def kernel(input, weight):
    Bm, T = input.shape
    D = weight.shape[1]
    B = Bm * T
    idx = jnp.transpose(input).reshape(B).astype(jnp.int32)
    out = _make_gather(B, D)(idx, weight)
    return jnp.transpose(out.reshape(T, Bm, D), (1, 0, 2))
